# baseline (speedup 1.0000x reference)
import jax
import jax.numpy as jnp
from jax import lax
from jax.experimental import pallas as pl
from jax.experimental.pallas import tpu as pltpu
from jax.experimental.pallas import tpu_sc as plsc

_MAX_DIST = 128
_D = 256
_L = 512
_T_ROWS = 2 * _MAX_DIST + 1

_NC = 2
_NS = 16

_DH = _D // 2
_T8_ROWS = _T_ROWS + 7
_FLAT = 208
_W_ROWS = 1016
_NCLS = 16
_ROWS_PER_CLS = _L // _NCLS


def _body(t8_hbm, f0_hbm, f1_hbm, out_hbm, w_ref, sem):
    wid = lax.axis_index("s") * _NC + lax.axis_index("c")
    h = wid % 2
    c16 = wid // 2
    p = lax.rem(c16, 8)
    dst0 = jnp.where(c16 < 8, 368, 376)

    def al(x):
        return pl.multiple_of(x, 8)

    dh = pl.ds(h * _DH, _DH)

    fills = [
        pltpu.async_copy(t8_hbm.at[p, :, dh],
                         w_ref.at[pl.ds(al(dst0), _T8_ROWS)], sem),
        pltpu.async_copy(f0_hbm.at[pl.ds(0, _FLAT), dh],
                         w_ref.at[pl.ds(0, _FLAT)], sem),
        pltpu.async_copy(f0_hbm.at[pl.ds(0, 176), dh],
                         w_ref.at[pl.ds(al(dst0 - 176), 176)], sem),
        pltpu.async_copy(f1_hbm.at[pl.ds(0, _FLAT), dh],
                         w_ref.at[pl.ds(al(dst0 + _T8_ROWS), _FLAT)], sem),
        pltpu.async_copy(f1_hbm.at[pl.ds(0, 176), dh],
                         w_ref.at[pl.ds(_W_ROWS - 176, 176)], sem),
    ]
    for f in fills:
        f.wait()

    handles = []
    for t in range(_ROWS_PER_CLS):
        i = c16 + _NCLS * t
        q = (_L - _NCLS) - _NCLS * t
        handles.append(
            pltpu.async_copy(w_ref.at[pl.ds(q, _L // 2)],
                             out_hbm.at[i, pl.ds(0, _L // 2), dh],
                             sem))
        handles.append(
            pltpu.async_copy(w_ref.at[pl.ds(q + _L // 2, _L // 2)],
                             out_hbm.at[i, pl.ds(_L // 2, _L // 2), dh],
                             sem))
    for hd in handles:
        hd.wait()


@jax.jit
def _rpe(table):
    t0 = table[0]
    t256 = table[_T_ROWS - 1]
    base = jnp.concatenate([
        jnp.broadcast_to(t0, (7, _D)),
        table,
        jnp.broadcast_to(t256, (7, _D)),
    ])
    t8 = jnp.stack([
        lax.slice(base, (7 - p_, 0), (7 - p_ + _T8_ROWS, _D))
        for p_ in range(8)
    ])
    f0 = jnp.broadcast_to(t0, (_FLAT, _D))
    f1 = jnp.broadcast_to(t256, (_FLAT, _D))

    mesh = plsc.VectorSubcoreMesh(core_axis_name="c", subcore_axis_name="s")
    return pl.kernel(
        _body,
        out_type=jax.ShapeDtypeStruct((_L, _L, _D), jnp.float32),
        mesh=mesh,
        scratch_types=[
            pltpu.VMEM((_W_ROWS, _DH), jnp.float32),
            pltpu.SemaphoreType.DMA,
        ],
        compiler_params=pltpu.CompilerParams(use_tc_tiling_on_sc=True),
    )(t8, f0, f1)


def kernel(seq_len, table):
    return _rpe(table)
